# baseline (speedup 1.0000x reference)
import functools

import jax
import jax.numpy as jnp
from jax import lax
from jax.experimental import pallas as pl
from jax.experimental.pallas import tpu as pltpu
from jax.experimental.pallas import tpu_sc as plsc


def _gather_rows(x2, heads_flat, seq):
    rows, cols = x2.shape
    info = plsc.get_sparse_core_info()
    ncores, nsub = info.num_cores, info.num_subcores
    nw = ncores * nsub
    rows_per_w = rows // nw
    chunk = rows_per_w
    while chunk * cols + rows_per_w > 120000:
        chunk //= 2
    n_chunks = rows_per_w // chunk
    in_kernel_offset = seq % rows_per_w == 0
    mesh = plsc.VectorSubcoreMesh(core_axis_name="c", subcore_axis_name="s")

    @functools.partial(
        pl.kernel,
        mesh=mesh,
        out_type=jax.ShapeDtypeStruct((rows, cols), x2.dtype),
        scratch_types=[
            pltpu.VMEM((chunk,), jnp.int32),
            pltpu.VMEM((chunk, cols), x2.dtype),
            pltpu.SemaphoreType.DMA,
        ],
    )
    def gk(x_hbm, idx_hbm, out_hbm, idx_v, rows_v, sem):
        wid = lax.axis_index("s") * ncores + lax.axis_index("c")
        for c in range(n_chunks):
            base = wid * rows_per_w + c * chunk
            pltpu.sync_copy(idx_hbm.at[pl.ds(base, chunk)], idx_v)
            if in_kernel_offset:
                off = (base // seq) * seq
                for i in range(chunk // 16):
                    sl = pl.ds(i * 16, 16)
                    idx_v[sl] = idx_v[sl] + off
            pltpu.async_copy(x_hbm.at[idx_v], rows_v, sem).wait()
            pltpu.sync_copy(rows_v, out_hbm.at[pl.ds(base, chunk)])

    if not in_kernel_offset:
        bsz = rows // seq
        offs = (jnp.arange(bsz, dtype=jnp.int32) * seq)[:, None]
        heads_flat = (heads_flat.reshape(bsz, seq) + offs).reshape(rows)
    return gk(x2, heads_flat)


def _unpack_halves(p, hdim):
    lo = lax.bitcast_convert_type((p & 0xFFFF).astype(jnp.uint16), jnp.bfloat16)
    hi = lax.bitcast_convert_type(
        lax.shift_right_logical(p, 16).astype(jnp.uint16), jnp.bfloat16)
    return jnp.concatenate([lo, hi], axis=1)


def _pack_halves(y, hdim):
    half = hdim // 2
    lo = lax.bitcast_convert_type(y[:, :half], jnp.uint16).astype(jnp.int32)
    hi = lax.bitcast_convert_type(y[:, half:], jnp.uint16).astype(jnp.int32)
    return lo | lax.shift_left(hi, 16)


def _layer(x2, h2, w_self_bf, w_head_bf, bias, mask2, layer, x_packed, h_packed,
           out_packed):
    rows = x2.shape[0]
    hdim = w_self_bf.shape[-1]
    bm = 256
    grid = (rows // bm,)
    x_cols = hdim // 2 if x_packed else hdim
    h_cols = hdim // 2 if h_packed else hdim
    out_cols = hdim // 2 if out_packed else hdim
    out_arr_dtype = jnp.int32 if out_packed else jnp.float32

    def body(x_ref, h_ref, ws_ref, wh_ref, b_ref, m_ref, o_ref):
        if x_packed:
            xb = _unpack_halves(x_ref[...], hdim)
        else:
            xb = x_ref[...].astype(jnp.bfloat16)
        if h_packed:
            hb = _unpack_halves(h_ref[...], hdim)
        else:
            hb = h_ref[...].astype(jnp.bfloat16)
        acc = jnp.dot(xb, ws_ref[0], preferred_element_type=jnp.float32)
        acc = acc + jnp.dot(hb, wh_ref[0], preferred_element_type=jnp.float32)
        acc = acc + b_ref[0]
        y = jnp.maximum(acc, 0.0) * m_ref[...]
        if out_packed:
            o_ref[...] = _pack_halves(y.astype(jnp.bfloat16), hdim)
        else:
            o_ref[...] = y

    return pl.pallas_call(
        body,
        grid=grid,
        in_specs=[
            pl.BlockSpec((bm, x_cols), lambda i: (i, 0)),
            pl.BlockSpec((bm, h_cols), lambda i: (i, 0)),
            pl.BlockSpec((1, hdim, hdim), lambda i: (layer, 0, 0)),
            pl.BlockSpec((1, hdim, hdim), lambda i: (layer, 0, 0)),
            pl.BlockSpec((1, 1, hdim), lambda i: (layer, 0, 0)),
            pl.BlockSpec((bm, 1), lambda i: (i, 0)),
        ],
        out_specs=pl.BlockSpec((bm, out_cols), lambda i: (i, 0)),
        out_shape=jax.ShapeDtypeStruct((rows, out_cols), out_arr_dtype),
    )(x2, h2, w_self_bf, w_head_bf, bias, mask2)


def kernel(hidden_states, attention_mask, heads, rels, W_self, W_head, b):
    del rels
    bsz, seq, hdim = hidden_states.shape
    rows = bsz * seq
    heads_flat = heads.astype(jnp.int32).reshape(rows)
    mask2 = attention_mask.reshape(rows, 1)
    num_layers = W_self.shape[0]
    ws_bf = W_self.astype(jnp.bfloat16)
    wh_bf = W_head.astype(jnp.bfloat16)
    b3 = b.reshape(num_layers, 1, hdim)
    x0 = hidden_states.reshape(rows, hdim)
    h2 = _gather_rows(x0, heads_flat, seq)
    half = hdim // 2
    x0b = x0.astype(jnp.bfloat16)
    lo0 = lax.bitcast_convert_type(x0b[:, :half], jnp.uint16).astype(jnp.int32)
    hi0 = lax.bitcast_convert_type(x0b[:, half:], jnp.uint16).astype(jnp.int32)
    x2 = lo0 | lax.shift_left(hi0, 16)
    for l in range(num_layers):
        if l > 0:
            h2 = _gather_rows(x2, heads_flat, seq)
        x2 = _layer(x2, h2, ws_bf, wh_bf, b3, mask2, l,
                    x_packed=True, h_packed=l > 0,
                    out_packed=l < num_layers - 1)
    return x2.reshape(bsz, seq, hdim)


# speedup vs baseline: 1.3027x; 1.3027x over previous
import functools

import jax
import jax.numpy as jnp
from jax import lax
from jax.experimental import pallas as pl
from jax.experimental.pallas import tpu as pltpu
from jax.experimental.pallas import tpu_sc as plsc


def _gather_rows(x2, heads_flat, seq):
    rows, cols = x2.shape
    info = plsc.get_sparse_core_info()
    ncores, nsub = info.num_cores, info.num_subcores
    nw = ncores * nsub
    rows_per_w = rows // nw
    chunk = rows_per_w
    while chunk * cols + rows_per_w > 120000:
        chunk //= 2
    n_chunks = rows_per_w // chunk
    in_kernel_offset = seq % rows_per_w == 0
    mesh = plsc.VectorSubcoreMesh(core_axis_name="c", subcore_axis_name="s")

    @functools.partial(
        pl.kernel,
        mesh=mesh,
        out_type=jax.ShapeDtypeStruct((rows, cols), x2.dtype),
        scratch_types=[
            pltpu.VMEM((chunk,), jnp.int32),
            pltpu.VMEM((chunk, cols), x2.dtype),
            pltpu.SemaphoreType.DMA,
        ],
    )
    def gk(x_hbm, idx_hbm, out_hbm, idx_v, rows_v, sem):
        wid = lax.axis_index("s") * ncores + lax.axis_index("c")
        for c in range(n_chunks):
            base = wid * rows_per_w + c * chunk
            pltpu.sync_copy(idx_hbm.at[pl.ds(base, chunk)], idx_v)
            if in_kernel_offset:
                off = (base // seq) * seq
                for i in range(chunk // 16):
                    sl = pl.ds(i * 16, 16)
                    idx_v[sl] = idx_v[sl] + off
            pltpu.async_copy(x_hbm.at[idx_v], rows_v, sem).wait()
            pltpu.sync_copy(rows_v, out_hbm.at[pl.ds(base, chunk)])

    if not in_kernel_offset:
        bsz = rows // seq
        offs = (jnp.arange(bsz, dtype=jnp.int32) * seq)[:, None]
        heads_flat = (heads_flat.reshape(bsz, seq) + offs).reshape(rows)
    return gk(x2, heads_flat)


def _unpack_halves(p, hdim):
    lo = lax.bitcast_convert_type((p & 0xFFFF).astype(jnp.uint16), jnp.bfloat16)
    hi = lax.bitcast_convert_type(
        lax.shift_right_logical(p, 16).astype(jnp.uint16), jnp.bfloat16)
    return jnp.concatenate([lo, hi], axis=1)


def _pack_halves(y, hdim):
    half = hdim // 2
    lo = lax.bitcast_convert_type(y[:, :half], jnp.uint16).astype(jnp.int32)
    hi = lax.bitcast_convert_type(y[:, half:], jnp.uint16).astype(jnp.int32)
    return lo | lax.shift_left(hi, 16)


def _layer(x2, h2, w_self_bf, w_head_bf, bias, mask2, layer, x_packed, h_packed,
           out_packed):
    rows = x2.shape[0]
    hdim = w_self_bf.shape[-1]
    bm = 512
    grid = (rows // bm,)
    x_cols = hdim // 2 if x_packed else hdim
    h_cols = hdim // 2 if h_packed else hdim
    out_cols = hdim // 2 if out_packed else hdim
    out_arr_dtype = jnp.int32 if out_packed else jnp.float32

    def body(x_ref, h_ref, ws_ref, wh_ref, b_ref, m_ref, o_ref):
        if x_packed:
            xb = _unpack_halves(x_ref[...], hdim)
        else:
            xb = x_ref[...].astype(jnp.bfloat16)
        if h_packed:
            hb = _unpack_halves(h_ref[...], hdim)
        else:
            hb = h_ref[...].astype(jnp.bfloat16)
        acc = jnp.dot(xb, ws_ref[0], preferred_element_type=jnp.float32)
        acc = acc + jnp.dot(hb, wh_ref[0], preferred_element_type=jnp.float32)
        acc = acc + b_ref[0]
        y = jnp.maximum(acc, 0.0) * m_ref[...]
        if out_packed:
            o_ref[...] = _pack_halves(y.astype(jnp.bfloat16), hdim)
        else:
            o_ref[...] = y

    return pl.pallas_call(
        body,
        grid=grid,
        in_specs=[
            pl.BlockSpec((bm, x_cols), lambda i: (i, 0)),
            pl.BlockSpec((bm, h_cols), lambda i: (i, 0)),
            pl.BlockSpec((1, hdim, hdim), lambda i: (layer, 0, 0)),
            pl.BlockSpec((1, hdim, hdim), lambda i: (layer, 0, 0)),
            pl.BlockSpec((1, 1, hdim), lambda i: (layer, 0, 0)),
            pl.BlockSpec((bm, 1), lambda i: (i, 0)),
        ],
        out_specs=pl.BlockSpec((bm, out_cols), lambda i: (i, 0)),
        out_shape=jax.ShapeDtypeStruct((rows, out_cols), out_arr_dtype),
    )(x2, h2, w_self_bf, w_head_bf, bias, mask2)


def kernel(hidden_states, attention_mask, heads, rels, W_self, W_head, b):
    del rels
    bsz, seq, hdim = hidden_states.shape
    rows = bsz * seq
    heads_flat = heads.astype(jnp.int32).reshape(rows)
    mask2 = attention_mask.reshape(rows, 1)
    num_layers = W_self.shape[0]
    ws_bf = W_self.astype(jnp.bfloat16)
    wh_bf = W_head.astype(jnp.bfloat16)
    b3 = b.reshape(num_layers, 1, hdim)
    x2 = hidden_states.reshape(rows, hdim)
    for l in range(num_layers):
        h2 = _gather_rows(x2, heads_flat, seq)
        x2 = _layer(x2, h2, ws_bf, wh_bf, b3, mask2, l,
                    x_packed=l > 0, h_packed=l > 0,
                    out_packed=l < num_layers - 1)
    return x2.reshape(bsz, seq, hdim)


# speedup vs baseline: 1.3884x; 1.0658x over previous
import functools

import jax
import jax.numpy as jnp
from jax import lax
from jax.experimental import pallas as pl
from jax.experimental.pallas import tpu as pltpu
from jax.experimental.pallas import tpu_sc as plsc


def _gather_rows(x2, heads_flat, seq):
    rows, cols = x2.shape
    info = plsc.get_sparse_core_info()
    ncores, nsub = info.num_cores, info.num_subcores
    nw = ncores * nsub
    rows_per_w = rows // nw
    chunk = rows_per_w
    while chunk * cols + rows_per_w > 120000:
        chunk //= 2
    n_chunks = rows_per_w // chunk
    in_kernel_offset = seq % rows_per_w == 0
    mesh = plsc.VectorSubcoreMesh(core_axis_name="c", subcore_axis_name="s")

    @functools.partial(
        pl.kernel,
        mesh=mesh,
        out_type=jax.ShapeDtypeStruct((rows, cols), x2.dtype),
        scratch_types=[
            pltpu.VMEM((chunk,), jnp.int32),
            pltpu.VMEM((chunk, cols), x2.dtype),
            pltpu.SemaphoreType.DMA,
        ],
    )
    def gk(x_hbm, idx_hbm, out_hbm, idx_v, rows_v, sem):
        wid = lax.axis_index("s") * ncores + lax.axis_index("c")
        for c in range(n_chunks):
            base = wid * rows_per_w + c * chunk
            pltpu.sync_copy(idx_hbm.at[pl.ds(base, chunk)], idx_v)
            if in_kernel_offset:
                off = (base // seq) * seq
                for i in range(chunk // 16):
                    sl = pl.ds(i * 16, 16)
                    idx_v[sl] = idx_v[sl] + off
            pltpu.async_copy(x_hbm.at[idx_v], rows_v, sem).wait()
            pltpu.sync_copy(rows_v, out_hbm.at[pl.ds(base, chunk)])

    if not in_kernel_offset:
        bsz = rows // seq
        offs = (jnp.arange(bsz, dtype=jnp.int32) * seq)[:, None]
        heads_flat = (heads_flat.reshape(bsz, seq) + offs).reshape(rows)
    return gk(x2, heads_flat)


def _unpack_halves(p, hdim):
    lo = lax.bitcast_convert_type((p & 0xFFFF).astype(jnp.uint16), jnp.bfloat16)
    hi = lax.bitcast_convert_type(
        lax.shift_right_logical(p, 16).astype(jnp.uint16), jnp.bfloat16)
    return jnp.concatenate([lo, hi], axis=1)


def _pack_halves(y, hdim):
    half = hdim // 2
    lo = lax.bitcast_convert_type(y[:, :half], jnp.uint16).astype(jnp.int32)
    hi = lax.bitcast_convert_type(y[:, half:], jnp.uint16).astype(jnp.int32)
    return lo | lax.shift_left(hi, 16)


def _layer(x2, h2, w_self_bf, w_head_bf, bias, mask2, layer, x_packed, h_packed,
           out_packed):
    rows = x2.shape[0]
    hdim = w_self_bf.shape[-1]
    bm = 1024
    grid = (rows // bm,)
    x_cols = hdim // 2 if x_packed else hdim
    h_cols = hdim // 2 if h_packed else hdim
    out_cols = hdim // 2 if out_packed else hdim
    out_arr_dtype = jnp.int32 if out_packed else jnp.float32

    def body(x_ref, h_ref, ws_ref, wh_ref, b_ref, m_ref, o_ref):
        if x_packed:
            xb = _unpack_halves(x_ref[...], hdim)
        else:
            xb = x_ref[...].astype(jnp.bfloat16)
        if h_packed:
            hb = _unpack_halves(h_ref[...], hdim)
        else:
            hb = h_ref[...].astype(jnp.bfloat16)
        acc = jnp.dot(xb, ws_ref[0], preferred_element_type=jnp.float32)
        acc = acc + jnp.dot(hb, wh_ref[0], preferred_element_type=jnp.float32)
        acc = acc + b_ref[0]
        y = jnp.maximum(acc, 0.0) * m_ref[...]
        if out_packed:
            o_ref[...] = _pack_halves(y.astype(jnp.bfloat16), hdim)
        else:
            o_ref[...] = y

    return pl.pallas_call(
        body,
        grid=grid,
        in_specs=[
            pl.BlockSpec((bm, x_cols), lambda i: (i, 0)),
            pl.BlockSpec((bm, h_cols), lambda i: (i, 0)),
            pl.BlockSpec((1, hdim, hdim), lambda i: (layer, 0, 0)),
            pl.BlockSpec((1, hdim, hdim), lambda i: (layer, 0, 0)),
            pl.BlockSpec((1, 1, hdim), lambda i: (layer, 0, 0)),
            pl.BlockSpec((bm, 1), lambda i: (i, 0)),
        ],
        out_specs=pl.BlockSpec((bm, out_cols), lambda i: (i, 0)),
        out_shape=jax.ShapeDtypeStruct((rows, out_cols), out_arr_dtype),
    )(x2, h2, w_self_bf, w_head_bf, bias, mask2)


def kernel(hidden_states, attention_mask, heads, rels, W_self, W_head, b):
    del rels
    bsz, seq, hdim = hidden_states.shape
    rows = bsz * seq
    heads_flat = heads.astype(jnp.int32).reshape(rows)
    mask2 = attention_mask.reshape(rows, 1)
    num_layers = W_self.shape[0]
    ws_bf = W_self.astype(jnp.bfloat16)
    wh_bf = W_head.astype(jnp.bfloat16)
    b3 = b.reshape(num_layers, 1, hdim)
    x2 = hidden_states.reshape(rows, hdim)
    for l in range(num_layers):
        h2 = _gather_rows(x2, heads_flat, seq)
        x2 = _layer(x2, h2, ws_bf, wh_bf, b3, mask2, l,
                    x_packed=l > 0, h_packed=l > 0,
                    out_packed=l < num_layers - 1)
    return x2.reshape(bsz, seq, hdim)
